# baseline (speedup 1.0000x reference)
import jax
import jax.numpy as jnp
from jax.experimental import pallas as pl
from jax.experimental.pallas import tpu as pltpu

_N_TOK = 8192
_DIM = 2048
_N_EXPERTS = 64
_TOPK = 8
_N_GROUPS = 8
_TOPK_GROUPS = 4
_GROUP_SIZE = _N_EXPERTS // _N_GROUPS
_ROUTE_SCALE = 2.5
_BLK = 1024
_NEG = -1e30


def _top2_sum(sg):
    m1, m2 = sg[:, :4, :], None
    a, b = sg[:, :4, :], sg[:, 4:, :]
    m1 = jnp.maximum(a, b)
    m2 = jnp.minimum(a, b)
    for half in (2, 1):
        a1, b1 = m1[:, :half, :], m1[:, half:, :]
        a2, b2 = m2[:, :half, :], m2[:, half:, :]
        m2 = jnp.maximum(jnp.minimum(a1, b1), jnp.maximum(a2, b2))
        m1 = jnp.maximum(a1, b1)
    return (m1 + m2)[:, 0, :]


def _route(logits, bias):
    blk = logits.shape[0]
    lt = logits.T
    orig = jax.nn.sigmoid(lt)
    s = orig + bias

    sg = s.reshape(_N_GROUPS, _GROUP_SIZE, blk)
    gs = _top2_sum(sg)

    gi = jax.lax.broadcasted_iota(jnp.int32, (_N_GROUPS, _N_GROUPS, 1), 0)
    gj = jax.lax.broadcasted_iota(jnp.int32, (_N_GROUPS, _N_GROUPS, 1), 1)
    tri = gj < gi
    ga = gs[:, None, :]
    gb = gs[None, :, :]
    beats = (gb > ga) | ((gb == ga) & tri)
    rank = jnp.sum(beats.astype(jnp.int32), axis=1)
    keep = (rank < _TOPK_GROUPS).astype(jnp.float32)
    keep_e = jnp.broadcast_to(
        keep[:, None, :],
        (_N_GROUPS, _GROUP_SIZE, blk)).reshape(_N_EXPERTS, blk)
    masked = s * keep_e

    row = jax.lax.broadcasted_iota(jnp.int32, (_N_EXPERTS, blk), 0)
    work = masked
    w_rows = []
    i_rows = []
    for _ in range(_TOPK):
        m = jnp.max(work, axis=0, keepdims=True)
        a = jnp.min(jnp.where(work == m, row, _N_EXPERTS),
                    axis=0, keepdims=True)
        sel = row == a
        i_rows.append(a)
        w_rows.append(jnp.sum(jnp.where(sel, orig, 0.0), axis=0,
                              keepdims=True))
        work = jnp.where(sel, _NEG, work)
    w_t = jnp.concatenate(w_rows, axis=0)
    i_t = jnp.concatenate(i_rows, axis=0)
    w_n = w_t / jnp.sum(w_t, axis=0, keepdims=True) * _ROUTE_SCALE
    return w_n, i_t


def _gate_kernel(xa_ref, xb_ref, wt_ref, bias_ref,
                 wa_ref, ia_ref, wb_ref, ib_ref):
    wt = wt_ref[...]
    bias = bias_ref[...]
    la = jnp.dot(xa_ref[...], wt, preferred_element_type=jnp.float32)
    w_a, i_a = _route(la, bias)
    wa_ref[...] = w_a.T
    ia_ref[...] = i_a.T
    lb = jnp.dot(xb_ref[...], wt, preferred_element_type=jnp.float32)
    w_b, i_b = _route(lb, bias)
    wb_ref[...] = w_b.T
    ib_ref[...] = i_b.T


def kernel(x, token_mask, weight, e_score_correction_bias):
    del token_mask
    n = x.shape[0]
    half_blocks = n // (2 * _BLK)
    wt = weight.T
    bias = e_score_correction_bias.reshape(_N_EXPERTS, 1)
    grid = (half_blocks,)
    w_a, i_a, w_b, i_b = pl.pallas_call(
        _gate_kernel,
        grid=grid,
        in_specs=[
            pl.BlockSpec((_BLK, _DIM), lambda i: (i, 0)),
            pl.BlockSpec((_BLK, _DIM),
                         lambda i: (i + half_blocks, 0)),
            pl.BlockSpec((_DIM, _N_EXPERTS), lambda i: (0, 0)),
            pl.BlockSpec((_N_EXPERTS, 1), lambda i: (0, 0)),
        ],
        out_specs=[
            pl.BlockSpec((_BLK, _TOPK), lambda i: (i, 0)),
            pl.BlockSpec((_BLK, _TOPK), lambda i: (i, 0)),
            pl.BlockSpec((_BLK, _TOPK), lambda i: (i, 0)),
            pl.BlockSpec((_BLK, _TOPK), lambda i: (i, 0)),
        ],
        out_shape=[
            jax.ShapeDtypeStruct((n // 2, _TOPK), jnp.float32),
            jax.ShapeDtypeStruct((n // 2, _TOPK), jnp.int32),
            jax.ShapeDtypeStruct((n // 2, _TOPK), jnp.float32),
            jax.ShapeDtypeStruct((n // 2, _TOPK), jnp.int32),
        ],
        compiler_params=pltpu.CompilerParams(
            dimension_semantics=("parallel",)),
    )(x, x, wt, bias)
    weights = jnp.concatenate([w_a, w_b], axis=0)
    indices = jnp.concatenate([i_a, i_b], axis=0)
    return weights.astype(x.dtype), indices


# speedup vs baseline: 1.0450x; 1.0450x over previous
import jax
import jax.numpy as jnp
from jax.experimental import pallas as pl
from jax.experimental.pallas import tpu as pltpu

_N_TOK = 8192
_DIM = 2048
_N_EXPERTS = 64
_TOPK = 8
_N_GROUPS = 8
_TOPK_GROUPS = 4
_GROUP_SIZE = _N_EXPERTS // _N_GROUPS
_ROUTE_SCALE = 2.5
_BLK = 2048
_NEG = -1e30


def _top2_sum(sg):
    m1, m2 = sg[:, :4, :], None
    a, b = sg[:, :4, :], sg[:, 4:, :]
    m1 = jnp.maximum(a, b)
    m2 = jnp.minimum(a, b)
    for half in (2, 1):
        a1, b1 = m1[:, :half, :], m1[:, half:, :]
        a2, b2 = m2[:, :half, :], m2[:, half:, :]
        m2 = jnp.maximum(jnp.minimum(a1, b1), jnp.maximum(a2, b2))
        m1 = jnp.maximum(a1, b1)
    return (m1 + m2)[:, 0, :]


def _route(logits, bias):
    blk = logits.shape[0]
    lt = logits.T
    orig = jax.nn.sigmoid(lt)
    s = orig + bias

    sg = s.reshape(_N_GROUPS, _GROUP_SIZE, blk)
    gs = _top2_sum(sg)

    gi = jax.lax.broadcasted_iota(jnp.int32, (_N_GROUPS, _N_GROUPS, 1), 0)
    gj = jax.lax.broadcasted_iota(jnp.int32, (_N_GROUPS, _N_GROUPS, 1), 1)
    tri = gj < gi
    ga = gs[:, None, :]
    gb = gs[None, :, :]
    beats = (gb > ga) | ((gb == ga) & tri)
    rank = jnp.sum(beats.astype(jnp.int32), axis=1)
    keep = (rank < _TOPK_GROUPS).astype(jnp.float32)
    keep_e = jnp.broadcast_to(
        keep[:, None, :],
        (_N_GROUPS, _GROUP_SIZE, blk)).reshape(_N_EXPERTS, blk)
    masked = s * keep_e

    row = jax.lax.broadcasted_iota(jnp.int32, (_N_EXPERTS, blk), 0)
    work = masked
    w_rows = []
    i_rows = []
    for _ in range(_TOPK):
        m = jnp.max(work, axis=0, keepdims=True)
        a = jnp.min(jnp.where(work == m, row, _N_EXPERTS),
                    axis=0, keepdims=True)
        sel = row == a
        i_rows.append(a)
        w_rows.append(jnp.sum(jnp.where(sel, orig, 0.0), axis=0,
                              keepdims=True))
        work = jnp.where(sel, _NEG, work)
    w_t = jnp.concatenate(w_rows, axis=0)
    i_t = jnp.concatenate(i_rows, axis=0)
    w_n = w_t / jnp.sum(w_t, axis=0, keepdims=True) * _ROUTE_SCALE
    return w_n, i_t


def _gate_kernel(x_ref, wt_ref, bias_ref, w_out_ref, i_out_ref):
    logits = jnp.dot(x_ref[...], wt_ref[...],
                     preferred_element_type=jnp.float32)
    w_n, i_t = _route(logits, bias_ref[...])
    w_out_ref[...] = w_n.T
    i_out_ref[...] = i_t.T


def kernel(x, token_mask, weight, e_score_correction_bias):
    del token_mask
    n = x.shape[0]
    wt = weight.T
    bias = e_score_correction_bias.reshape(_N_EXPERTS, 1)
    grid = (n // _BLK,)
    weights, indices = pl.pallas_call(
        _gate_kernel,
        grid=grid,
        in_specs=[
            pl.BlockSpec((_BLK, _DIM), lambda i: (i, 0)),
            pl.BlockSpec((_DIM, _N_EXPERTS), lambda i: (0, 0)),
            pl.BlockSpec((_N_EXPERTS, 1), lambda i: (0, 0)),
        ],
        out_specs=[
            pl.BlockSpec((_BLK, _TOPK), lambda i: (i, 0)),
            pl.BlockSpec((_BLK, _TOPK), lambda i: (i, 0)),
        ],
        out_shape=[
            jax.ShapeDtypeStruct((n, _TOPK), jnp.float32),
            jax.ShapeDtypeStruct((n, _TOPK), jnp.int32),
        ],
        compiler_params=pltpu.CompilerParams(
            dimension_semantics=("parallel",)),
    )(x, wt, bias)
    return weights.astype(x.dtype), indices
